# baseline (speedup 1.0000x reference)
import jax
import jax.numpy as jnp
from jax import lax
from jax.experimental import pallas as pl
from jax.experimental.pallas import tpu as pltpu
from jax.experimental.pallas import tpu_sc as plsc

_WD = 0.01
_B = 16384
_D = 16
_NC = 2
_NS = 16
_NW = _NC * _NS
_BPW = _B // _NW
_CHUNK = 128
_NCHUNK = _BPW // _CHUNK


def _sc_body(w_hbm, h_hbm, u_hbm, i_hbm, j_hbm, out_hbm,
             u_v, i_v, j_v, ue_v, ie_v, je_v, res_v, sem):
    wid = lax.axis_index("s") * _NC + lax.axis_index("c")
    base = wid * _BPW

    pltpu.sync_copy(u_hbm.at[pl.ds(base, _BPW)], u_v)
    pltpu.sync_copy(i_hbm.at[pl.ds(base, _BPW)], i_v)
    pltpu.sync_copy(j_hbm.at[pl.ds(base, _BPW)], j_v)

    copies = []
    for k in range(_NCHUNK):
        sl = pl.ds(k * _CHUNK, _CHUNK)
        copies.append(pltpu.async_copy(w_hbm.at[u_v.at[sl]], ue_v.at[sl], sem))
        copies.append(pltpu.async_copy(h_hbm.at[i_v.at[sl]], ie_v.at[sl], sem))
        copies.append(pltpu.async_copy(h_hbm.at[j_v.at[sl]], je_v.at[sl], sem))
    for c in copies:
        c.wait()

    iota16 = lax.iota(jnp.int32, 16)
    cols = [jnp.full((16,), d, jnp.int32) for d in range(_D)]

    def block(t, carry):
        ls_acc, reg_acc = carry
        rows = t * 16 + iota16
        x = jnp.zeros((16,), jnp.float32)
        reg = reg_acc
        for d in range(_D):
            cu = plsc.load_gather(ue_v, [rows, cols[d]])
            ci = plsc.load_gather(ie_v, [rows, cols[d]])
            cj = plsc.load_gather(je_v, [rows, cols[d]])
            x = x + cu * (ci - cj)
            reg = reg + cu * cu + ci * ci + cj * cj
        w = jnp.exp(-jnp.abs(x))
        t_ = w / (w + 2.0)
        t2 = t_ * t_
        poly = 1.0 + t2 * (1.0 / 3.0 + t2 * (1.0 / 5.0 + t2 * (
            1.0 / 7.0 + t2 * (1.0 / 9.0 + t2 * (1.0 / 11.0)))))
        ls = ls_acc + jnp.minimum(x, 0.0) - 2.0 * t_ * poly
        return (ls, reg)

    zero = jnp.zeros((16,), jnp.float32)
    ls_acc, reg_acc = lax.fori_loop(0, _BPW // 16, block, (zero, zero))

    res_v[...] = _WD * reg_acc - ls_acc
    pltpu.sync_copy(res_v, out_hbm.at[wid])


@jax.jit
def _bpr_partials(w, h, u, i, j):
    mesh = plsc.VectorSubcoreMesh(core_axis_name="c", subcore_axis_name="s")
    return pl.kernel(
        _sc_body,
        out_type=jax.ShapeDtypeStruct((_NW, 16), jnp.float32),
        mesh=mesh,
        compiler_params=pltpu.CompilerParams(
            needs_layout_passes=False, use_tc_tiling_on_sc=False),
        scratch_types=[
            pltpu.VMEM((_BPW,), jnp.int32),
            pltpu.VMEM((_BPW,), jnp.int32),
            pltpu.VMEM((_BPW,), jnp.int32),
            pltpu.VMEM((_BPW, _D), jnp.float32),
            pltpu.VMEM((_BPW, _D), jnp.float32),
            pltpu.VMEM((_BPW, _D), jnp.float32),
            pltpu.VMEM((16,), jnp.float32),
            pltpu.SemaphoreType.DMA,
        ],
    )(w, h, u, i, j)


def kernel(W, H, u, i, i_pop, j, j_pop):
    del i_pop, j_pop
    partials = _bpr_partials(
        W, H,
        u.astype(jnp.int32), i.astype(jnp.int32), j.astype(jnp.int32))
    return jnp.sum(partials)


# speedup vs baseline: 3.7121x; 3.7121x over previous
import jax
import jax.numpy as jnp
from jax import lax
from jax.experimental import pallas as pl
from jax.experimental.pallas import tpu as pltpu
from jax.experimental.pallas import tpu_sc as plsc

_WD = 0.01
_B = 16384
_D = 16
_NC = 2
_NS = 16
_NW = _NC * _NS
_BPW = _B // _NW
_N = 1000000
_SLAB = 2048
_TAIL0 = 999936
_MINI0 = 999424
_CLAMP = 997888
_LCAP = 2048
_GOUT = _B * _D + 16


def _p1_body(wt, ht, u_h, i_h, j_h, wtail, htail, gu, gi, gj,
             idx_v, slab0, slab1, ru, bu, ri, bi, rj, bj,
             slr, slb, tmp, wtl_v, htl_v, drain_v,
             sem0, sem1, semd):
    w = lax.axis_index("s") * _NC + lax.axis_index("c")
    start = w * 30720 + jnp.minimum(w, 8) * 2048
    nsub = 15 + jnp.where(w < 8, 1, 0)
    rend = start + nsub * _SLAB + jnp.where(w == 31, 512, 0)

    iota16 = lax.iota(jnp.int32, 16)
    zeros16 = jnp.zeros((16,), jnp.int32)
    bdump16 = jnp.full((16,), _B, jnp.int32)

    pltpu.sync_copy(wtail, wtl_v)
    pltpu.sync_copy(htail, htl_v)

    def build(src_h, tail_v, rlist, blist, gout):
        pltpu.sync_copy(src_h, idx_v)

        def chunk(t, cnt):
            v = idx_v[pl.ds(t * 16, 16)]
            m = (v >= start) & (v < rend)
            plsc.store_compressed(rlist.at[pl.ds(cnt, 16)], v, mask=m)
            b = t * 16 + iota16
            plsc.store_compressed(blist.at[pl.ds(cnt, 16)], b, mask=m)
            pc = plsc.all_reduce_population_count(m)[0]
            mt = v >= _TAIL0
            pt = plsc.all_reduce_population_count(mt)[0]

            @pl.when((w == 31) & (pt > 0))
            def _():
                mi = mt.astype(jnp.int32)
                for lane in range(16):
                    @pl.when(mi[lane] > 0)
                    def _():
                        pltpu.sync_copy(
                            tail_v.at[v[lane] - _TAIL0],
                            gout.at[pl.ds((t * 16 + lane) * _D, _D)])

            return cnt + pc

        cnt = lax.fori_loop(0, _B // 16, chunk, 0)
        rlist[pl.ds(cnt, 16)] = zeros16 + start
        blist[pl.ds(cnt, 16)] = bdump16
        return cnt

    cnt_u = build(u_h, wtl_v, ru, bu, gu)
    cnt_i = build(i_h, htl_v, ri, bi, gi)
    cnt_j = build(j_h, htl_v, rj, bj, gj)

    cols = [jnp.full((16,), d, jnp.int32) for d in range(_D)]

    def process(slab, r0, width, jobs, fb):
        for (rlist, blist, cnt, gout) in jobs:
            nq = (cnt >> 4) + 1

            def qc(q, c2):
                rv = rlist[pl.ds(q * 16, 16)]
                bv = blist[pl.ds(q * 16, 16)]
                rl = rv - r0
                m = (rl >= 0) & (rl < width)
                plsc.store_compressed(slr.at[pl.ds(c2, 16)], rl, mask=m)
                plsc.store_compressed(slb.at[pl.ds(c2, 16)], bv, mask=m)
                return c2 + plsc.all_reduce_population_count(m)[0]

            c2 = lax.fori_loop(0, nq, qc, 0)
            slr[pl.ds(c2, 16)] = zeros16
            slb[pl.ds(c2, 16)] = bdump16
            nd = (c2 >> 4) + 1

            def dense(qd, fb):
                fired, drained = fb
                slot = (fired >> 4) & 3
                ndr = jnp.where(fired - drained >= 64, 16, 0)

                def dr(_, c):
                    pltpu.make_async_copy(
                        gu.at[pl.ds(0, 16)], drain_v, semd).wait()
                    return c

                lax.fori_loop(0, ndr, dr, 0)
                rv2 = slr[pl.ds(qd * 16, 16)]
                bv2 = slb[pl.ds(qd * 16, 16)]
                for d in range(_D):
                    vd = plsc.load_gather(slab, [cols[d], rv2])
                    plsc.store_scatter(tmp.at[slot], [iota16, cols[d]], vd)
                for lane in range(16):
                    pltpu.async_copy(
                        tmp.at[slot].at[lane],
                        gout.at[pl.ds(bv2[lane] * _D, _D)], semd)
                return (fired + 16, drained + ndr)

            fb = lax.fori_loop(0, nd, dense, fb)
        return fb

    def serve(table, jobs, fb):
        def r0_of(s):
            return jnp.minimum(start + s * _SLAB, _CLAMP)

        def fire(s, buf, sem):
            return pltpu.async_copy(
                table.at[:, pl.ds(r0_of(s), _SLAB)], buf, sem)

        fire(0, slab0, sem0)

        def pair(k, fb):
            fire(2 * k + 1, slab1, sem1)
            pltpu.make_async_copy(
                table.at[:, pl.ds(0, _SLAB)], slab0, sem0).wait()
            fb = process(slab0, r0_of(2 * k), _SLAB, jobs, fb)

            @pl.when(k < 7)
            def _():
                fire(2 * k + 2, slab0, sem0)

            @pl.when(k == 7)
            def _():
                pltpu.async_copy(
                    table.at[:, pl.ds(_MINI0, 512)],
                    slab0.at[:, pl.ds(0, 512)], sem0)

            pltpu.make_async_copy(
                table.at[:, pl.ds(0, _SLAB)], slab1, sem1).wait()
            return process(slab1, r0_of(2 * k + 1), _SLAB, jobs, fb)

        fb = lax.fori_loop(0, 8, pair, fb)
        pltpu.make_async_copy(
            table.at[:, pl.ds(0, 512)], slab0.at[:, pl.ds(0, 512)],
            sem0).wait()
        return process(slab0, _MINI0, 512, jobs, fb)

    fb = serve(wt, [(ru, bu, cnt_u, gu)], (0, 0))
    fb = serve(ht, [(ri, bi, cnt_i, gi), (rj, bj, cnt_j, gj)], fb)
    fired, drained = fb

    def dr(_, c):
        pltpu.make_async_copy(gu.at[pl.ds(0, 16)], drain_v, semd).wait()
        return c

    lax.fori_loop(0, fired - drained, dr, 0)


def _p2_body(gu_h, gi_h, gj_h, out_hbm, gu_v, gi_v, gj_v, res_v):
    wid = lax.axis_index("s") * _NC + lax.axis_index("c")
    base = wid * _BPW * _D

    pltpu.sync_copy(gu_h.at[pl.ds(base, _BPW * _D)], gu_v)
    pltpu.sync_copy(gi_h.at[pl.ds(base, _BPW * _D)], gi_v)
    pltpu.sync_copy(gj_h.at[pl.ds(base, _BPW * _D)], gj_v)

    iota16 = lax.iota(jnp.int32, 16)
    stride16 = iota16 * _D

    def block(t, carry):
        ls_acc, reg_acc = carry
        x = jnp.zeros((16,), jnp.float32)
        reg = reg_acc
        for d in range(_D):
            idxv = stride16 + (t * (16 * _D) + d)
            cu = plsc.load_gather(gu_v, [idxv])
            ci = plsc.load_gather(gi_v, [idxv])
            cj = plsc.load_gather(gj_v, [idxv])
            x = x + cu * (ci - cj)
            reg = reg + cu * cu + ci * ci + cj * cj
        w = jnp.exp(-jnp.abs(x))
        t_ = w / (w + 2.0)
        t2 = t_ * t_
        poly = 1.0 + t2 * (1.0 / 3.0 + t2 * (1.0 / 5.0 + t2 * (
            1.0 / 7.0 + t2 * (1.0 / 9.0 + t2 * (1.0 / 11.0)))))
        ls = ls_acc + jnp.minimum(x, 0.0) - 2.0 * t_ * poly
        return (ls, reg)

    zero = jnp.zeros((16,), jnp.float32)
    ls_acc, reg_acc = lax.fori_loop(0, _BPW // 16, block, (zero, zero))

    res_v[...] = _WD * reg_acc - ls_acc
    pltpu.sync_copy(res_v, out_hbm.at[pl.ds(wid * 16, 16)])


@jax.jit
def _bpr(wt, ht, u, i, j, wtail, htail):
    mesh = plsc.VectorSubcoreMesh(core_axis_name="c", subcore_axis_name="s")
    gu, gi, gj = pl.kernel(
        _p1_body,
        out_type=[jax.ShapeDtypeStruct((_GOUT,), jnp.float32)] * 3,
        mesh=mesh,
        compiler_params=pltpu.CompilerParams(needs_layout_passes=False),
        scratch_types=[
            pltpu.VMEM((_B,), jnp.int32),
            pltpu.VMEM((_D, _SLAB), jnp.float32),
            pltpu.VMEM((_D, _SLAB), jnp.float32),
            pltpu.VMEM((_LCAP + 16,), jnp.int32),
            pltpu.VMEM((_LCAP + 16,), jnp.int32),
            pltpu.VMEM((_LCAP + 16,), jnp.int32),
            pltpu.VMEM((_LCAP + 16,), jnp.int32),
            pltpu.VMEM((_LCAP + 16,), jnp.int32),
            pltpu.VMEM((_LCAP + 16,), jnp.int32),
            pltpu.VMEM((_LCAP + 32,), jnp.int32),
            pltpu.VMEM((_LCAP + 32,), jnp.int32),
            pltpu.VMEM((4, 16, _D), jnp.float32),
            pltpu.VMEM((64, _D), jnp.float32),
            pltpu.VMEM((64, _D), jnp.float32),
            pltpu.VMEM((16,), jnp.float32),
            pltpu.SemaphoreType.DMA,
            pltpu.SemaphoreType.DMA,
            pltpu.SemaphoreType.DMA,
        ],
    )(wt, ht, u, i, j, wtail, htail)
    partials = pl.kernel(
        _p2_body,
        out_type=jax.ShapeDtypeStruct((_NW * 16,), jnp.float32),
        mesh=mesh,
        compiler_params=pltpu.CompilerParams(needs_layout_passes=False),
        scratch_types=[
            pltpu.VMEM((_BPW * _D,), jnp.float32),
            pltpu.VMEM((_BPW * _D,), jnp.float32),
            pltpu.VMEM((_BPW * _D,), jnp.float32),
            pltpu.VMEM((16,), jnp.float32),
        ],
    )(gu, gi, gj)
    return jnp.sum(partials)


def kernel(W, H, u, i, i_pop, j, j_pop):
    del i_pop, j_pop
    return _bpr(W.T, H.T,
                u.astype(jnp.int32), i.astype(jnp.int32),
                j.astype(jnp.int32),
                W[_TAIL0:], H[_TAIL0:])


# speedup vs baseline: 4.6002x; 1.2392x over previous
import jax
import jax.numpy as jnp
from jax import lax
from jax.experimental import pallas as pl
from jax.experimental.pallas import tpu as pltpu
from jax.experimental.pallas import tpu_sc as plsc

_WD = 0.01
_B = 16384
_D = 16
_NC = 2
_NS = 16
_NW = _NC * _NS
_BPW = _B // _NW
_N = 1000000
_SLAB = 2048
_TAIL0 = 999936
_MINI0 = 999424
_CLAMP = 997888
_LCAP = 2048
_GOUT = _B * _D + 16


def _p1_body(wt, ht, u_h, i_h, j_h, wtail, htail, gu, gi, gj,
             idx_v, slab0, slab1, ru, bu, ri, bi, rj, bj,
             slr, slb, tmp, wtl_v, htl_v, drain_v,
             sem0, sem1, semd):
    w = lax.axis_index("s") * _NC + lax.axis_index("c")
    start = w * 30720 + jnp.minimum(w, 8) * 2048
    nsub = 15 + jnp.where(w < 8, 1, 0)
    rend = jnp.where(w == 31, _N, start + nsub * _SLAB)

    iota16 = lax.iota(jnp.int32, 16)
    zeros16 = jnp.zeros((16,), jnp.int32)
    bdump16 = jnp.full((16,), _B, jnp.int32)

    pltpu.sync_copy(wtail, wtl_v)
    pltpu.sync_copy(htail, htl_v)

    def build(src_h, rlist, blist):
        pltpu.sync_copy(src_h, idx_v)

        def chunk(t, cnt):
            v = idx_v[pl.ds(t * 16, 16)]
            m = (v >= start) & (v < rend)
            plsc.store_compressed(rlist.at[pl.ds(cnt, 16)], v, mask=m)
            b = t * 16 + iota16
            plsc.store_compressed(blist.at[pl.ds(cnt, 16)], b, mask=m)
            return cnt + plsc.all_reduce_population_count(m)[0]

        cnt = lax.fori_loop(0, _B // 16, chunk, 0)
        rlist[pl.ds(cnt, 16)] = zeros16 + start
        blist[pl.ds(cnt, 16)] = bdump16
        return cnt

    cnt_u = build(u_h, ru, bu)
    cnt_i = build(i_h, ri, bi)
    cnt_j = build(j_h, rj, bj)

    cols = [jnp.full((16,), d, jnp.int32) for d in range(_D)]

    def process(slab, r0, width, jobs, fb):
        for (rlist, blist, cnt, gout) in jobs:
            nq = (cnt >> 4) + 1

            def qc(q, c2):
                rv = rlist[pl.ds(q * 16, 16)]
                bv = blist[pl.ds(q * 16, 16)]
                rl = rv - r0
                m = (rl >= 0) & (rl < width)
                plsc.store_compressed(slr.at[pl.ds(c2, 16)], rl, mask=m)
                plsc.store_compressed(slb.at[pl.ds(c2, 16)], bv, mask=m)
                return c2 + plsc.all_reduce_population_count(m)[0]

            c2 = lax.fori_loop(0, nq, qc, 0)
            slr[pl.ds(c2, 16)] = zeros16
            slb[pl.ds(c2, 16)] = bdump16
            nd = (c2 >> 4) + 1

            def dense(qd, fb):
                fired, drained = fb
                slot = (fired >> 4) & 3
                ndr = jnp.where(fired - drained >= 64, 16, 0)

                def dr(_, c):
                    pltpu.make_async_copy(
                        gu.at[pl.ds(0, 16)], drain_v, semd).wait()
                    return c

                lax.fori_loop(0, ndr, dr, 0)
                rv2 = slr[pl.ds(qd * 16, 16)]
                bv2 = slb[pl.ds(qd * 16, 16)]
                for d in range(_D):
                    vd = plsc.load_gather(slab, [cols[d], rv2])
                    plsc.store_scatter(tmp.at[slot], [iota16, cols[d]], vd)
                for lane in range(16):
                    pltpu.async_copy(
                        tmp.at[slot].at[lane],
                        gout.at[pl.ds(bv2[lane] * _D, _D)], semd)
                return (fired + 16, drained + ndr)

            fb = lax.fori_loop(0, nd, dense, fb)
        return fb

    def serve(table, tail_v, jobs, fb):
        def r0_of(s):
            return jnp.minimum(start + s * _SLAB, _CLAMP)

        def fire(s, buf, sem):
            return pltpu.async_copy(
                table.at[:, pl.ds(r0_of(s), _SLAB)], buf, sem)

        fire(0, slab0, sem0)

        def pair(k, fb):
            fire(2 * k + 1, slab1, sem1)
            pltpu.make_async_copy(
                table.at[:, pl.ds(0, _SLAB)], slab0, sem0).wait()
            fb = process(slab0, r0_of(2 * k), _SLAB, jobs, fb)

            @pl.when(k < 7)
            def _():
                fire(2 * k + 2, slab0, sem0)

            @pl.when(k == 7)
            def _():
                pltpu.async_copy(
                    table.at[:, pl.ds(_MINI0, 512)],
                    slab0.at[:, pl.ds(0, 512)], sem0)

            pltpu.make_async_copy(
                table.at[:, pl.ds(0, _SLAB)], slab1, sem1).wait()
            return process(slab1, r0_of(2 * k + 1), _SLAB, jobs, fb)

        fb = lax.fori_loop(0, 8, pair, fb)
        pltpu.make_async_copy(
            table.at[:, pl.ds(0, 512)], slab0.at[:, pl.ds(0, 512)],
            sem0).wait()
        fb = process(slab0, _MINI0, 512, jobs, fb)
        return process(tail_v, _TAIL0, 64, jobs, fb)

    fb = serve(wt, wtl_v, [(ru, bu, cnt_u, gu)], (0, 0))
    fb = serve(ht, htl_v, [(ri, bi, cnt_i, gi), (rj, bj, cnt_j, gj)], fb)
    fired, drained = fb

    def dr(_, c):
        pltpu.make_async_copy(gu.at[pl.ds(0, 16)], drain_v, semd).wait()
        return c

    lax.fori_loop(0, fired - drained, dr, 0)


def _p2_body(gu_h, gi_h, gj_h, out_hbm, gu_v, gi_v, gj_v, res_v):
    wid = lax.axis_index("s") * _NC + lax.axis_index("c")
    base = wid * _BPW * _D

    pltpu.sync_copy(gu_h.at[pl.ds(base, _BPW * _D)], gu_v)
    pltpu.sync_copy(gi_h.at[pl.ds(base, _BPW * _D)], gi_v)
    pltpu.sync_copy(gj_h.at[pl.ds(base, _BPW * _D)], gj_v)

    iota16 = lax.iota(jnp.int32, 16)
    stride16 = iota16 * _D

    def block(t, carry):
        ls_acc, reg_acc = carry
        x = jnp.zeros((16,), jnp.float32)
        reg = reg_acc
        for d in range(_D):
            idxv = stride16 + (t * (16 * _D) + d)
            cu = plsc.load_gather(gu_v, [idxv])
            ci = plsc.load_gather(gi_v, [idxv])
            cj = plsc.load_gather(gj_v, [idxv])
            x = x + cu * (ci - cj)
            reg = reg + cu * cu + ci * ci + cj * cj
        w = jnp.exp(-jnp.abs(x))
        t_ = w / (w + 2.0)
        t2 = t_ * t_
        poly = 1.0 + t2 * (1.0 / 3.0 + t2 * (1.0 / 5.0 + t2 * (
            1.0 / 7.0 + t2 * (1.0 / 9.0 + t2 * (1.0 / 11.0)))))
        ls = ls_acc + jnp.minimum(x, 0.0) - 2.0 * t_ * poly
        return (ls, reg)

    zero = jnp.zeros((16,), jnp.float32)
    ls_acc, reg_acc = lax.fori_loop(0, _BPW // 16, block, (zero, zero))

    res_v[...] = _WD * reg_acc - ls_acc
    pltpu.sync_copy(res_v, out_hbm.at[pl.ds(wid * 16, 16)])


@jax.jit
def _bpr(wt, ht, u, i, j, wtail, htail):
    mesh = plsc.VectorSubcoreMesh(core_axis_name="c", subcore_axis_name="s")
    gu, gi, gj = pl.kernel(
        _p1_body,
        out_type=[jax.ShapeDtypeStruct((_GOUT,), jnp.float32)] * 3,
        mesh=mesh,
        compiler_params=pltpu.CompilerParams(needs_layout_passes=False),
        scratch_types=[
            pltpu.VMEM((_B,), jnp.int32),
            pltpu.VMEM((_D, _SLAB), jnp.float32),
            pltpu.VMEM((_D, _SLAB), jnp.float32),
            pltpu.VMEM((_LCAP + 16,), jnp.int32),
            pltpu.VMEM((_LCAP + 16,), jnp.int32),
            pltpu.VMEM((_LCAP + 16,), jnp.int32),
            pltpu.VMEM((_LCAP + 16,), jnp.int32),
            pltpu.VMEM((_LCAP + 16,), jnp.int32),
            pltpu.VMEM((_LCAP + 16,), jnp.int32),
            pltpu.VMEM((_LCAP + 32,), jnp.int32),
            pltpu.VMEM((_LCAP + 32,), jnp.int32),
            pltpu.VMEM((4, 16, _D), jnp.float32),
            pltpu.VMEM((_D, 64), jnp.float32),
            pltpu.VMEM((_D, 64), jnp.float32),
            pltpu.VMEM((16,), jnp.float32),
            pltpu.SemaphoreType.DMA,
            pltpu.SemaphoreType.DMA,
            pltpu.SemaphoreType.DMA,
        ],
    )(wt, ht, u, i, j, wtail, htail)
    partials = pl.kernel(
        _p2_body,
        out_type=jax.ShapeDtypeStruct((_NW * 16,), jnp.float32),
        mesh=mesh,
        compiler_params=pltpu.CompilerParams(needs_layout_passes=False),
        scratch_types=[
            pltpu.VMEM((_BPW * _D,), jnp.float32),
            pltpu.VMEM((_BPW * _D,), jnp.float32),
            pltpu.VMEM((_BPW * _D,), jnp.float32),
            pltpu.VMEM((16,), jnp.float32),
        ],
    )(gu, gi, gj)
    return jnp.sum(partials)


def kernel(W, H, u, i, i_pop, j, j_pop):
    del i_pop, j_pop
    Wt = W.T
    Ht = H.T
    return _bpr(Wt, Ht,
                u.astype(jnp.int32), i.astype(jnp.int32),
                j.astype(jnp.int32),
                Wt[:, _TAIL0:], Ht[:, _TAIL0:])


# speedup vs baseline: 4.7933x; 1.0420x over previous
import jax
import jax.numpy as jnp
from jax import lax
from jax.experimental import pallas as pl
from jax.experimental.pallas import tpu as pltpu
from jax.experimental.pallas import tpu_sc as plsc

_WD = 0.01
_B = 16384
_D = 16
_NC = 2
_NS = 16
_NW = _NC * _NS
_BPW = _B // _NW
_N = 1000000
_SLAB = 2048
_TAIL0 = 999936
_MINI0 = 999424
_CLAMP = 997888
_LCAP = 1536
_GOUT = _B * _D + 16


def _p1_body(wt, ht, u_h, i_h, j_h, wtail, htail, gu, gi, gj,
             u_v, i_v, j_v, slab0, slab1, ru, bu, ri, bi, rj, bj,
             slr, slb, tmp, wtl_v, htl_v, drain_v,
             sem0, sem1, semd):
    w = lax.axis_index("s") * _NC + lax.axis_index("c")
    start = w * 30720 + jnp.minimum(w, 8) * 2048
    nsub = 15 + jnp.where(w < 8, 1, 0)
    rend = jnp.where(w == 31, _N, start + nsub * _SLAB)

    iota16 = lax.iota(jnp.int32, 16)
    zeros16 = jnp.zeros((16,), jnp.int32)
    bdump16 = jnp.full((16,), _B, jnp.int32)

    pltpu.sync_copy(wtail, wtl_v)
    pltpu.sync_copy(htail, htl_v)

    def half(h, cnts):
        pltpu.sync_copy(u_h.at[pl.ds(h * 8192, 8192)], u_v)
        pltpu.sync_copy(i_h.at[pl.ds(h * 8192, 8192)], i_v)
        pltpu.sync_copy(j_h.at[pl.ds(h * 8192, 8192)], j_v)

        def chunk(t, cnts):
            b = (h * 512 + t) * 16 + iota16
            out = []
            for v_ref, rlist, blist, cnt in zip(
                    (u_v, i_v, j_v), (ru, ri, rj), (bu, bi, bj), cnts):
                v = v_ref[pl.ds(t * 16, 16)]
                m = (v >= start) & (v < rend)
                plsc.store_compressed(rlist.at[pl.ds(cnt, 16)], v, mask=m)
                plsc.store_compressed(blist.at[pl.ds(cnt, 16)], b, mask=m)
                out.append(cnt + plsc.all_reduce_population_count(m)[0])
            return tuple(out)

        return lax.fori_loop(0, 512, chunk, cnts)

    cnt_u, cnt_i, cnt_j = lax.fori_loop(0, 2, half, (0, 0, 0))
    for rlist, blist, cnt in ((ru, bu, cnt_u), (ri, bi, cnt_i),
                              (rj, bj, cnt_j)):
        rlist[pl.ds(cnt, 16)] = zeros16 + start
        blist[pl.ds(cnt, 16)] = bdump16

    cols = [jnp.full((16,), d, jnp.int32) for d in range(_D)]

    def process(slab, r0, width, jobs, fb):
        for (rlist, blist, cnt, gout) in jobs:
            nq = (cnt >> 4) + 1

            def qc(q, c2):
                rv = rlist[pl.ds(q * 16, 16)]
                bv = blist[pl.ds(q * 16, 16)]
                rl = rv - r0
                m = (rl >= 0) & (rl < width)
                plsc.store_compressed(slr.at[pl.ds(c2, 16)], rl, mask=m)
                plsc.store_compressed(slb.at[pl.ds(c2, 16)], bv, mask=m)
                return c2 + plsc.all_reduce_population_count(m)[0]

            c2 = lax.fori_loop(0, nq, qc, 0)
            slr[pl.ds(c2, 16)] = zeros16
            slb[pl.ds(c2, 16)] = bdump16
            nd = (c2 >> 4) + 1

            def dense(qd, fb):
                fired, drained = fb
                slot = (fired >> 4) & 3
                ndr = jnp.where(fired - drained >= 64, 16, 0)

                def dr(_, c):
                    pltpu.make_async_copy(
                        gu.at[pl.ds(0, 16)], drain_v, semd).wait()
                    return c

                lax.fori_loop(0, ndr, dr, 0)
                rv2 = slr[pl.ds(qd * 16, 16)]
                bv2 = slb[pl.ds(qd * 16, 16)]
                for d in range(_D):
                    vd = plsc.load_gather(slab, [cols[d], rv2])
                    plsc.store_scatter(tmp.at[slot], [iota16, cols[d]], vd)
                for lane in range(16):
                    pltpu.async_copy(
                        tmp.at[slot].at[lane],
                        gout.at[pl.ds(bv2[lane] * _D, _D)], semd)
                return (fired + 16, drained + ndr)

            fb = lax.fori_loop(0, nd, dense, fb)
        return fb

    def serve(table, tail_v, jobs, fb):
        def r0_of(s):
            return jnp.minimum(start + s * _SLAB, _CLAMP)

        def fire(s, buf, sem):
            return pltpu.async_copy(
                table.at[:, pl.ds(r0_of(s), _SLAB)], buf, sem)

        fire(0, slab0, sem0)

        def pair(k, fb):
            fire(2 * k + 1, slab1, sem1)
            pltpu.make_async_copy(
                table.at[:, pl.ds(0, _SLAB)], slab0, sem0).wait()
            fb = process(slab0, r0_of(2 * k), _SLAB, jobs, fb)

            @pl.when(k < 7)
            def _():
                fire(2 * k + 2, slab0, sem0)

            @pl.when(k == 7)
            def _():
                pltpu.async_copy(
                    table.at[:, pl.ds(_MINI0, 512)],
                    slab0.at[:, pl.ds(0, 512)], sem0)

            pltpu.make_async_copy(
                table.at[:, pl.ds(0, _SLAB)], slab1, sem1).wait()
            return process(slab1, r0_of(2 * k + 1), _SLAB, jobs, fb)

        fb = lax.fori_loop(0, 8, pair, fb)
        pltpu.make_async_copy(
            table.at[:, pl.ds(0, 512)], slab0.at[:, pl.ds(0, 512)],
            sem0).wait()
        fb = process(slab0, _MINI0, 512, jobs, fb)
        return process(tail_v, _TAIL0, 64, jobs, fb)

    fb = serve(wt, wtl_v, [(ru, bu, cnt_u, gu)], (0, 0))
    fb = serve(ht, htl_v, [(ri, bi, cnt_i, gi), (rj, bj, cnt_j, gj)], fb)
    fired, drained = fb

    def dr(_, c):
        pltpu.make_async_copy(gu.at[pl.ds(0, 16)], drain_v, semd).wait()
        return c

    lax.fori_loop(0, fired - drained, dr, 0)


def _p2_body(gu_h, gi_h, gj_h, out_hbm, gu_v, gi_v, gj_v, res_v):
    wid = lax.axis_index("s") * _NC + lax.axis_index("c")
    base = wid * _BPW * _D

    pltpu.sync_copy(gu_h.at[pl.ds(base, _BPW * _D)], gu_v)
    pltpu.sync_copy(gi_h.at[pl.ds(base, _BPW * _D)], gi_v)
    pltpu.sync_copy(gj_h.at[pl.ds(base, _BPW * _D)], gj_v)

    iota16 = lax.iota(jnp.int32, 16)
    stride16 = iota16 * _D

    def block(t, carry):
        ls_acc, reg_acc = carry
        x = jnp.zeros((16,), jnp.float32)
        reg = reg_acc
        for d in range(_D):
            idxv = stride16 + (t * (16 * _D) + d)
            cu = plsc.load_gather(gu_v, [idxv])
            ci = plsc.load_gather(gi_v, [idxv])
            cj = plsc.load_gather(gj_v, [idxv])
            x = x + cu * (ci - cj)
            reg = reg + cu * cu + ci * ci + cj * cj
        w = jnp.exp(-jnp.abs(x))
        t_ = w / (w + 2.0)
        t2 = t_ * t_
        poly = 1.0 + t2 * (1.0 / 3.0 + t2 * (1.0 / 5.0 + t2 * (
            1.0 / 7.0 + t2 * (1.0 / 9.0 + t2 * (1.0 / 11.0)))))
        ls = ls_acc + jnp.minimum(x, 0.0) - 2.0 * t_ * poly
        return (ls, reg)

    zero = jnp.zeros((16,), jnp.float32)
    ls_acc, reg_acc = lax.fori_loop(0, _BPW // 16, block, (zero, zero))

    res_v[...] = _WD * reg_acc - ls_acc
    pltpu.sync_copy(res_v, out_hbm.at[pl.ds(wid * 16, 16)])


@jax.jit
def _bpr(wt, ht, u, i, j, wtail, htail):
    mesh = plsc.VectorSubcoreMesh(core_axis_name="c", subcore_axis_name="s")
    gu, gi, gj = pl.kernel(
        _p1_body,
        out_type=[jax.ShapeDtypeStruct((_GOUT,), jnp.float32)] * 3,
        mesh=mesh,
        compiler_params=pltpu.CompilerParams(needs_layout_passes=False),
        scratch_types=[
            pltpu.VMEM((8192,), jnp.int32),
            pltpu.VMEM((8192,), jnp.int32),
            pltpu.VMEM((8192,), jnp.int32),
            pltpu.VMEM((_D, _SLAB), jnp.float32),
            pltpu.VMEM((_D, _SLAB), jnp.float32),
            pltpu.VMEM((_LCAP + 16,), jnp.int32),
            pltpu.VMEM((_LCAP + 16,), jnp.int32),
            pltpu.VMEM((_LCAP + 16,), jnp.int32),
            pltpu.VMEM((_LCAP + 16,), jnp.int32),
            pltpu.VMEM((_LCAP + 16,), jnp.int32),
            pltpu.VMEM((_LCAP + 16,), jnp.int32),
            pltpu.VMEM((_LCAP + 32,), jnp.int32),
            pltpu.VMEM((_LCAP + 32,), jnp.int32),
            pltpu.VMEM((4, 16, _D), jnp.float32),
            pltpu.VMEM((_D, 64), jnp.float32),
            pltpu.VMEM((_D, 64), jnp.float32),
            pltpu.VMEM((16,), jnp.float32),
            pltpu.SemaphoreType.DMA,
            pltpu.SemaphoreType.DMA,
            pltpu.SemaphoreType.DMA,
        ],
    )(wt, ht, u, i, j, wtail, htail)
    partials = pl.kernel(
        _p2_body,
        out_type=jax.ShapeDtypeStruct((_NW * 16,), jnp.float32),
        mesh=mesh,
        compiler_params=pltpu.CompilerParams(needs_layout_passes=False),
        scratch_types=[
            pltpu.VMEM((_BPW * _D,), jnp.float32),
            pltpu.VMEM((_BPW * _D,), jnp.float32),
            pltpu.VMEM((_BPW * _D,), jnp.float32),
            pltpu.VMEM((16,), jnp.float32),
        ],
    )(gu, gi, gj)
    return jnp.sum(partials)


def kernel(W, H, u, i, i_pop, j, j_pop):
    del i_pop, j_pop
    Wt = W.T
    Ht = H.T
    return _bpr(Wt, Ht,
                u.astype(jnp.int32), i.astype(jnp.int32),
                j.astype(jnp.int32),
                Wt[:, _TAIL0:], Ht[:, _TAIL0:])


# speedup vs baseline: 4.9721x; 1.0373x over previous
import jax
import jax.numpy as jnp
from jax import lax
from jax.experimental import pallas as pl
from jax.experimental.pallas import tpu as pltpu
from jax.experimental.pallas import tpu_sc as plsc

_WD = 0.01
_B = 16384
_D = 16
_NC = 2
_NS = 16
_NW = _NC * _NS
_BPW = _B // _NW
_N = 1000000
_SLAB = 2048
_TAIL0 = 999936
_MINI0 = 999424
_CLAMP = 997888
_LCAP = 1536
_GOUT = _B * _D + 16


def _p1_body(wt, ht, u_h, i_h, j_h, wtail, htail, gu, gi, gj,
             u_v, i_v, j_v, slab0, slab1, ru, bu, ri, bi, rj, bj,
             slr, slb, tmp, wtl_v, htl_v, drain_v,
             sem0, sem1, semd):
    w = lax.axis_index("s") * _NC + lax.axis_index("c")
    start = w * 30720 + jnp.minimum(w, 8) * 2048
    nsub = 15 + jnp.where(w < 8, 1, 0)
    rend = jnp.where(w == 31, _N, start + nsub * _SLAB)

    iota16 = lax.iota(jnp.int32, 16)
    zeros16 = jnp.zeros((16,), jnp.int32)
    bdump16 = jnp.full((16,), _B, jnp.int32)

    pltpu.sync_copy(wtail, wtl_v)
    pltpu.sync_copy(htail, htl_v)

    def half(h, cnts):
        pltpu.sync_copy(u_h.at[pl.ds(h * 8192, 8192)], u_v)
        pltpu.sync_copy(i_h.at[pl.ds(h * 8192, 8192)], i_v)
        pltpu.sync_copy(j_h.at[pl.ds(h * 8192, 8192)], j_v)

        def chunk(t, cnts):
            b = (h * 512 + t) * 16 + iota16
            out = []
            for v_ref, rlist, blist, cnt in zip(
                    (u_v, i_v, j_v), (ru, ri, rj), (bu, bi, bj), cnts):
                v = v_ref[pl.ds(t * 16, 16)]
                m = (v >= start) & (v < rend)
                plsc.store_compressed(rlist.at[pl.ds(cnt, 16)], v, mask=m)
                plsc.store_compressed(blist.at[pl.ds(cnt, 16)], b, mask=m)
                out.append(cnt + plsc.all_reduce_population_count(m)[0])
            return tuple(out)

        return lax.fori_loop(0, 512, chunk, cnts)

    cnt_u, cnt_i, cnt_j = lax.fori_loop(0, 2, half, (0, 0, 0))
    for rlist, blist, cnt in ((ru, bu, cnt_u), (ri, bi, cnt_i),
                              (rj, bj, cnt_j)):
        rlist[pl.ds(cnt, 16)] = zeros16 + start
        blist[pl.ds(cnt, 16)] = bdump16

    cols = [jnp.full((16,), d, jnp.int32) for d in range(_D)]

    def process(slab, r0, width, jobs, fb, active):
        for (rlist, blist, cnt, gout) in jobs:
            nq = jnp.where(active, (cnt >> 4) + 1, 0)

            def qc(q, c2):
                rv = rlist[pl.ds(q * 16, 16)]
                bv = blist[pl.ds(q * 16, 16)]
                rl = rv - r0
                m = (rl >= 0) & (rl < width)
                plsc.store_compressed(slr.at[pl.ds(c2, 16)], rl, mask=m)
                plsc.store_compressed(slb.at[pl.ds(c2, 16)], bv, mask=m)
                return c2 + plsc.all_reduce_population_count(m)[0]

            c2 = lax.fori_loop(0, nq, qc, 0)
            slr[pl.ds(c2, 16)] = zeros16
            slb[pl.ds(c2, 16)] = bdump16
            nd = jnp.where(c2 > 0, (c2 >> 4) + 1, 0)

            def dense(qd, fb):
                fired, drained = fb
                slot = (fired >> 4) & 3
                ndr = jnp.where(fired - drained >= 64, 16, 0)

                def dr(_, c):
                    pltpu.make_async_copy(
                        gu.at[pl.ds(0, 16)], drain_v, semd).wait()
                    return c

                lax.fori_loop(0, ndr, dr, 0)
                rv2 = slr[pl.ds(qd * 16, 16)]
                bv2 = slb[pl.ds(qd * 16, 16)]
                for d in range(_D):
                    vd = plsc.load_gather(slab, [cols[d], rv2])
                    plsc.store_scatter(tmp.at[slot], [iota16, cols[d]], vd)
                for lane in range(16):
                    pltpu.async_copy(
                        tmp.at[slot].at[lane],
                        gout.at[pl.ds(bv2[lane] * _D, _D)], semd)
                return (fired + 16, drained + ndr)

            fb = lax.fori_loop(0, nd, dense, fb)
        return fb

    def serve(table, tail_v, jobs, fb):
        def r0_of(s):
            return jnp.minimum(start + s * _SLAB, _CLAMP)

        def fire(s, buf, sem):
            return pltpu.async_copy(
                table.at[:, pl.ds(r0_of(s), _SLAB)], buf, sem)

        fire(0, slab0, sem0)

        def pair(k, fb):
            fire(2 * k + 1, slab1, sem1)
            pltpu.make_async_copy(
                table.at[:, pl.ds(0, _SLAB)], slab0, sem0).wait()
            r0u = start + (2 * k) * _SLAB
            fb = process(slab0, r0_of(2 * k), _SLAB, jobs, fb,
                         r0u < rend)

            @pl.when(k < 7)
            def _():
                fire(2 * k + 2, slab0, sem0)

            @pl.when(k == 7)
            def _():
                pltpu.async_copy(
                    table.at[:, pl.ds(_MINI0, 512)],
                    slab0.at[:, pl.ds(0, 512)], sem0)

            pltpu.make_async_copy(
                table.at[:, pl.ds(0, _SLAB)], slab1, sem1).wait()
            return process(slab1, r0_of(2 * k + 1), _SLAB, jobs, fb,
                           start + (2 * k + 1) * _SLAB < rend)

        fb = lax.fori_loop(0, 8, pair, fb)
        pltpu.make_async_copy(
            table.at[:, pl.ds(0, 512)], slab0.at[:, pl.ds(0, 512)],
            sem0).wait()
        fb = process(slab0, _MINI0, 512, jobs, fb, _MINI0 < rend)
        return process(tail_v, _TAIL0, 64, jobs, fb, _TAIL0 < rend)

    fb = serve(wt, wtl_v, [(ru, bu, cnt_u, gu)], (0, 0))
    fb = serve(ht, htl_v, [(ri, bi, cnt_i, gi), (rj, bj, cnt_j, gj)], fb)
    fired, drained = fb

    def dr(_, c):
        pltpu.make_async_copy(gu.at[pl.ds(0, 16)], drain_v, semd).wait()
        return c

    lax.fori_loop(0, fired - drained, dr, 0)


def _p2_body(gu_h, gi_h, gj_h, out_hbm, gu_v, gi_v, gj_v, res_v):
    wid = lax.axis_index("s") * _NC + lax.axis_index("c")
    base = wid * _BPW * _D

    pltpu.sync_copy(gu_h.at[pl.ds(base, _BPW * _D)], gu_v)
    pltpu.sync_copy(gi_h.at[pl.ds(base, _BPW * _D)], gi_v)
    pltpu.sync_copy(gj_h.at[pl.ds(base, _BPW * _D)], gj_v)

    iota16 = lax.iota(jnp.int32, 16)
    stride16 = iota16 * _D

    def block(t, carry):
        ls_acc, reg_acc = carry
        x = jnp.zeros((16,), jnp.float32)
        reg = reg_acc
        for d in range(_D):
            idxv = stride16 + (t * (16 * _D) + d)
            cu = plsc.load_gather(gu_v, [idxv])
            ci = plsc.load_gather(gi_v, [idxv])
            cj = plsc.load_gather(gj_v, [idxv])
            x = x + cu * (ci - cj)
            reg = reg + cu * cu + ci * ci + cj * cj
        w = jnp.exp(-jnp.abs(x))
        t_ = w / (w + 2.0)
        t2 = t_ * t_
        poly = 1.0 + t2 * (1.0 / 3.0 + t2 * (1.0 / 5.0 + t2 * (
            1.0 / 7.0 + t2 * (1.0 / 9.0 + t2 * (1.0 / 11.0)))))
        ls = ls_acc + jnp.minimum(x, 0.0) - 2.0 * t_ * poly
        return (ls, reg)

    zero = jnp.zeros((16,), jnp.float32)
    ls_acc, reg_acc = lax.fori_loop(0, _BPW // 16, block, (zero, zero))

    res_v[...] = _WD * reg_acc - ls_acc
    pltpu.sync_copy(res_v, out_hbm.at[pl.ds(wid * 16, 16)])


@jax.jit
def _bpr(wt, ht, u, i, j, wtail, htail):
    mesh = plsc.VectorSubcoreMesh(core_axis_name="c", subcore_axis_name="s")
    gu, gi, gj = pl.kernel(
        _p1_body,
        out_type=[jax.ShapeDtypeStruct((_GOUT,), jnp.float32)] * 3,
        mesh=mesh,
        compiler_params=pltpu.CompilerParams(needs_layout_passes=False),
        scratch_types=[
            pltpu.VMEM((8192,), jnp.int32),
            pltpu.VMEM((8192,), jnp.int32),
            pltpu.VMEM((8192,), jnp.int32),
            pltpu.VMEM((_D, _SLAB), jnp.float32),
            pltpu.VMEM((_D, _SLAB), jnp.float32),
            pltpu.VMEM((_LCAP + 16,), jnp.int32),
            pltpu.VMEM((_LCAP + 16,), jnp.int32),
            pltpu.VMEM((_LCAP + 16,), jnp.int32),
            pltpu.VMEM((_LCAP + 16,), jnp.int32),
            pltpu.VMEM((_LCAP + 16,), jnp.int32),
            pltpu.VMEM((_LCAP + 16,), jnp.int32),
            pltpu.VMEM((_LCAP + 32,), jnp.int32),
            pltpu.VMEM((_LCAP + 32,), jnp.int32),
            pltpu.VMEM((4, 16, _D), jnp.float32),
            pltpu.VMEM((_D, 64), jnp.float32),
            pltpu.VMEM((_D, 64), jnp.float32),
            pltpu.VMEM((16,), jnp.float32),
            pltpu.SemaphoreType.DMA,
            pltpu.SemaphoreType.DMA,
            pltpu.SemaphoreType.DMA,
        ],
    )(wt, ht, u, i, j, wtail, htail)
    partials = pl.kernel(
        _p2_body,
        out_type=jax.ShapeDtypeStruct((_NW * 16,), jnp.float32),
        mesh=mesh,
        compiler_params=pltpu.CompilerParams(needs_layout_passes=False),
        scratch_types=[
            pltpu.VMEM((_BPW * _D,), jnp.float32),
            pltpu.VMEM((_BPW * _D,), jnp.float32),
            pltpu.VMEM((_BPW * _D,), jnp.float32),
            pltpu.VMEM((16,), jnp.float32),
        ],
    )(gu, gi, gj)
    return jnp.sum(partials)


def kernel(W, H, u, i, i_pop, j, j_pop):
    del i_pop, j_pop
    Wt = W.T
    Ht = H.T
    return _bpr(Wt, Ht,
                u.astype(jnp.int32), i.astype(jnp.int32),
                j.astype(jnp.int32),
                Wt[:, _TAIL0:], Ht[:, _TAIL0:])
